# baseline (speedup 1.0000x reference)
import functools

import jax
import jax.numpy as jnp
from jax import lax
from jax.experimental import pallas as pl
from jax.experimental.pallas import tpu as pltpu
from jax.experimental.pallas import tpu_sc as plsc

N = 2097152
D = 16
NB = N // 128
NC = 2
NS = 16
NW = NC * NS
C = 1024
TB = C // 128
G = C // 128
RW = N // NW
NCHUNK = RW // C
U = 8
PLANE = TB * 8 * 128
HPLANE = NB * 8 * 128

_mesh = plsc.VectorSubcoreMesh(core_axis_name="c", subcore_axis_name="s")


def _flat_consts():
    l = jax.lax.iota(jnp.int32, 16)
    a_c = l >> 3
    f_c = l & 7
    return a_c * PLANE + f_c * 128


@functools.partial(
    pl.kernel,
    mesh=_mesh,
    compiler_params=pltpu.CompilerParams(use_tc_tiling_on_sc=False, needs_layout_passes=False),
    out_type=jax.ShapeDtypeStruct((N, D), jnp.float32),
    scratch_types=[
        pltpu.VMEM((2 * PLANE,), jnp.float32),
        pltpu.VMEM((C, D), jnp.float32),
    ],
)
def _transpose_sc(xf_hbm, xrm_hbm, tbuf, rows_v):
    wid = lax.axis_index("s") * NC + lax.axis_index("c")
    b_base = wid * (NB // NW)
    base_c = _flat_consts()

    def chunk_body(j, carry):
        b0 = b_base + j * TB
        pltpu.sync_copy(xf_hbm.at[pl.ds(b0 * 1024, PLANE)],
                        tbuf.at[pl.ds(0, PLANE)])
        pltpu.sync_copy(xf_hbm.at[pl.ds(HPLANE + b0 * 1024, PLANE)],
                        tbuf.at[pl.ds(PLANE, PLANE)])

        def row_body(i, carry2):
            ibase = i * U
            for u in range(U):
                r = ibase + u
                off = ((r >> 7) << 10) + (r & 127)
                idxv = base_c + jnp.broadcast_to(off, (16,))
                rows_v[r] = plsc.load_gather(tbuf, [idxv])
            return carry2

        lax.fori_loop(0, C // U, row_body, 0)
        pltpu.sync_copy(rows_v, xrm_hbm.at[pl.ds(b0 * 128, C)])
        return carry

    lax.fori_loop(0, NB // NW // TB, chunk_body, 0)


@functools.partial(
    pl.kernel,
    mesh=_mesh,
    compiler_params=pltpu.CompilerParams(use_tc_tiling_on_sc=False, needs_layout_passes=False),
    out_type=jax.ShapeDtypeStruct((2 * HPLANE,), jnp.float32),
    scratch_types=[
        pltpu.VMEM((G, 128), jnp.int32),
        pltpu.VMEM((C, D), jnp.float32),
        pltpu.VMEM((C, D), jnp.float32),
        pltpu.VMEM((2 * PLANE,), jnp.float32),
        pltpu.VMEM((16,), jnp.float32),
        pltpu.SemaphoreType.DMA,
    ],
)
def _gather_shrink_sc(xrm_hbm, idx_hbm, t_hbm, outf_hbm,
                      idx_v, rows_v, x_v, trans_v, t_v, sem):
    wid = lax.axis_index("s") * NC + lax.axis_index("c")
    base = wid * RW
    base_c = _flat_consts()
    pltpu.sync_copy(t_hbm, t_v)
    tvec = t_v[...]
    signbit = jnp.broadcast_to(jnp.int32(-2147483648), (16,))
    zero = jnp.zeros((16,), jnp.float32)

    def chunk_body(j, carry):
        off = base + j * C
        pltpu.sync_copy(idx_hbm.at[pl.ds(wid * (RW // 128) + j * G, G)], idx_v)
        copies = []
        for g in range(G):
            copies.append(
                pltpu.async_copy(
                    xrm_hbm.at[idx_v.at[g]],
                    rows_v.at[pl.ds(g * 128, 128)],
                    sem,
                )
            )
        pltpu.sync_copy(xrm_hbm.at[pl.ds(off, C)], x_v)
        for cp in copies:
            cp.wait()

        def row_body(i, carry2):
            ibase = i * U
            for u in range(U):
                r = ibase + u
                gv = rows_v[r]
                xv = x_v[r]
                s = jnp.maximum(jnp.abs(gv) - tvec, 0.0)
                zb = plsc.bitcast(s, jnp.int32) | (
                    plsc.bitcast(xv, jnp.int32) & signbit)
                z = jnp.where(xv == 0.0, zero, plsc.bitcast(zb, jnp.float32))
                roff = ((r >> 7) << 10) + (r & 127)
                idxv = base_c + jnp.broadcast_to(roff, (16,))
                plsc.store_scatter(trans_v, [idxv], z)
            return carry2

        lax.fori_loop(0, C // U, row_body, 0)
        pltpu.sync_copy(trans_v.at[pl.ds(0, PLANE)],
                        outf_hbm.at[pl.ds(off * 8, PLANE)])
        pltpu.sync_copy(trans_v.at[pl.ds(PLANE, PLANE)],
                        outf_hbm.at[pl.ds(HPLANE + off * 8, PLANE)])
        return carry

    lax.fori_loop(0, NCHUNK, chunk_body, 0)


def kernel(x, rho, indices, thres):
    t = jax.nn.softplus(thres[0]) / rho[0]
    t16 = jnp.full((16,), t, dtype=jnp.float32)
    idx = indices.astype(jnp.int32).reshape(NB, 128)
    xf = x.transpose(1, 0).reshape(2, 8, NB, 128).transpose(0, 2, 1, 3).reshape(-1)
    xrm = _transpose_sc(xf)
    outf = _gather_shrink_sc(xrm, idx, t16)
    return (outf.reshape(2, NB, 8, 128).transpose(1, 3, 0, 2).reshape(N, D))


# speedup vs baseline: 1.1412x; 1.1412x over previous
import functools

import jax
import jax.numpy as jnp
from jax import lax
from jax.experimental import pallas as pl
from jax.experimental.pallas import tpu as pltpu
from jax.experimental.pallas import tpu_sc as plsc

N = 2097152
D = 16
NB = N // 128
NC = 2
NS = 16
NW = NC * NS
C = 1024
TB = C // 128
G = C // 128
RW = N // NW
NCHUNK = RW // C
U = 8
PLANE = TB * 8 * 128
HPLANE = NB * 8 * 128

_mesh = plsc.VectorSubcoreMesh(core_axis_name="c", subcore_axis_name="s")


def _flat_consts():
    l = jax.lax.iota(jnp.int32, 16)
    a_c = l >> 3
    f_c = l & 7
    return a_c * PLANE + f_c * 128


@functools.partial(
    pl.kernel,
    mesh=_mesh,
    compiler_params=pltpu.CompilerParams(use_tc_tiling_on_sc=False,
                                         needs_layout_passes=False),
    out_type=jax.ShapeDtypeStruct((N, D), jnp.float32),
    scratch_types=[
        pltpu.VMEM((2 * PLANE,), jnp.float32),
        pltpu.VMEM((2 * PLANE,), jnp.float32),
        pltpu.VMEM((C, D), jnp.float32),
        pltpu.VMEM((C, D), jnp.float32),
        pltpu.SemaphoreType.DMA,
        pltpu.SemaphoreType.DMA,
    ],
)
def _transpose_sc(xf_hbm, xrm_hbm, tb0, tb1, rv0, rv1, sin, sout):
    wid = lax.axis_index("s") * NC + lax.axis_index("c")
    b_base = wid * (NB // NW)
    base_c = _flat_consts()

    def in_descs(jc, tb):
        b0 = b_base + jc * TB
        return (
            pltpu.make_async_copy(xf_hbm.at[pl.ds(b0 * 1024, PLANE)],
                                  tb.at[pl.ds(0, PLANE)], sin),
            pltpu.make_async_copy(xf_hbm.at[pl.ds(HPLANE + b0 * 1024, PLANE)],
                                  tb.at[pl.ds(PLANE, PLANE)], sin),
        )

    def out_desc(jc, rv):
        b0 = b_base + jc * TB
        return pltpu.make_async_copy(rv, xrm_hbm.at[pl.ds(b0 * 128, C)], sout)

    for dsc in in_descs(0, tb0):
        dsc.start()
    for dsc in in_descs(1, tb1):
        dsc.start()

    def body(jj, carry):
        for b, tb, rv in ((0, tb0, rv0), (1, tb1, rv1)):
            j = jj * 2 + b
            for dsc in in_descs(j, tb):
                dsc.wait()

            @pl.when(jj > 0)
            def _():
                out_desc(j - 2, rv).wait()

            def tile_body(bb, carry2):
                iv0 = base_c + jnp.broadcast_to(bb * 1024, (16,))
                rb0 = bb * 128

                def r_body(ii, iv):
                    rbase = rb0 + ii * U
                    for u in range(U):
                        rv[rbase + u] = plsc.load_gather(tb, [iv])
                        iv = iv + 1
                    return iv

                lax.fori_loop(0, 128 // U, r_body, iv0)
                return carry2

            lax.fori_loop(0, TB, tile_body, 0)
            out_desc(j, rv).start()
            nj = jnp.where(j + 2 < NCHUNK, j + 2, 0)
            for dsc in in_descs(nj, tb):
                dsc.start()
        return carry

    lax.fori_loop(0, NCHUNK // 2, body, 0)
    for dsc in in_descs(0, tb0):
        dsc.wait()
    for dsc in in_descs(0, tb1):
        dsc.wait()
    out_desc(NCHUNK - 2, rv0).wait()
    out_desc(NCHUNK - 1, rv1).wait()


@functools.partial(
    pl.kernel,
    mesh=_mesh,
    compiler_params=pltpu.CompilerParams(use_tc_tiling_on_sc=False,
                                         needs_layout_passes=False),
    out_type=jax.ShapeDtypeStruct((2 * HPLANE,), jnp.float32),
    scratch_types=[
        pltpu.VMEM((G, 128), jnp.int32),
        pltpu.VMEM((G, 128), jnp.int32),
        pltpu.VMEM((C, D), jnp.float32),
        pltpu.VMEM((C, D), jnp.float32),
        pltpu.VMEM((C, D), jnp.float32),
        pltpu.VMEM((C, D), jnp.float32),
        pltpu.VMEM((2 * PLANE,), jnp.float32),
        pltpu.VMEM((2 * PLANE,), jnp.float32),
        pltpu.VMEM((16,), jnp.float32),
        pltpu.SemaphoreType.DMA,
        pltpu.SemaphoreType.DMA,
        pltpu.SemaphoreType.DMA,
        pltpu.SemaphoreType.DMA,
    ],
)
def _gather_shrink_sc(xrm_hbm, idx_hbm, t_hbm, outf_hbm,
                      iv0_, iv1_, rv0, rv1, xv0, xv1, tv0, tv1, t_v,
                      sidx, sg, sx, sout):
    wid = lax.axis_index("s") * NC + lax.axis_index("c")
    base = wid * RW
    base_c = _flat_consts()
    pltpu.sync_copy(t_hbm, t_v)
    tvec = t_v[...]
    signbit = jnp.broadcast_to(jnp.int32(-2147483648), (16,))
    zero = jnp.zeros((16,), jnp.float32)

    def idx_desc(jc, ivb):
        return pltpu.make_async_copy(
            idx_hbm.at[pl.ds(wid * (RW // 128) + jc * G, G)], ivb, sidx)

    def gather_descs(jc, ivb, rvb):
        return [
            pltpu.make_async_copy(xrm_hbm.at[ivb.at[g]],
                                  rvb.at[pl.ds(g * 128, 128)], sg)
            for g in range(G)
        ]

    def x_desc(jc, xvb):
        return pltpu.make_async_copy(
            xrm_hbm.at[pl.ds(base + jc * C, C)], xvb, sx)

    def out_descs(jc, tvb):
        off8 = (base + jc * C) * 8
        return (
            pltpu.make_async_copy(tvb.at[pl.ds(0, PLANE)],
                                  outf_hbm.at[pl.ds(off8, PLANE)], sout),
            pltpu.make_async_copy(tvb.at[pl.ds(PLANE, PLANE)],
                                  outf_hbm.at[pl.ds(HPLANE + off8, PLANE)],
                                  sout),
        )

    bufs = ((iv0_, rv0, xv0, tv0), (iv1_, rv1, xv1, tv1))

    idx_desc(0, iv0_).start()
    idx_desc(0, iv0_).wait()
    for dsc in gather_descs(0, iv0_, rv0):
        dsc.start()
    x_desc(0, xv0).start()
    idx_desc(1, iv1_).start()

    def body(jj, carry):
        for b in (0, 1):
            ivb, rvb, xvb, tvb = bufs[b]
            nivb, nrvb, nxvb, _ = bufs[1 - b]
            j = jj * 2 + b
            for dsc in gather_descs(j, ivb, rvb):
                dsc.wait()
            x_desc(j, xvb).wait()

            @pl.when(jj > 0)
            def _():
                for dsc in out_descs(j - 2, tvb):
                    dsc.wait()

            def tile_body(bb, carry2):
                iv = base_c + jnp.broadcast_to(bb * 1024, (16,))
                rb0 = bb * 128

                def r_body(ii, ivv):
                    rbase = rb0 + ii * U
                    for u in range(U):
                        r = rbase + u
                        gv = rvb[r]
                        xv = xvb[r]
                        s = jnp.maximum(jnp.abs(gv) - tvec, 0.0)
                        zb = plsc.bitcast(s, jnp.int32) | (
                            plsc.bitcast(xv, jnp.int32) & signbit)
                        z = jnp.where(xv == 0.0, zero,
                                      plsc.bitcast(zb, jnp.float32))
                        plsc.store_scatter(tvb, [ivv], z)
                        ivv = ivv + 1
                    return ivv

                lax.fori_loop(0, 128 // U, r_body, iv)
                return carry2

            lax.fori_loop(0, TB, tile_body, 0)
            for dsc in out_descs(j, tvb):
                dsc.start()
            nj = jnp.where(j + 1 < NCHUNK, j + 1, 0)
            idx_desc(nj, nivb).wait()
            for dsc in gather_descs(nj, nivb, nrvb):
                dsc.start()
            x_desc(nj, nxvb).start()
            nj2 = jnp.where(j + 2 < NCHUNK, j + 2, 0)
            idx_desc(nj2, ivb).start()
        return carry

    lax.fori_loop(0, NCHUNK // 2, body, 0)
    for dsc in gather_descs(0, iv0_, rv0):
        dsc.wait()
    x_desc(0, xv0).wait()
    idx_desc(0, iv1_).wait()
    for dsc in out_descs(NCHUNK - 2, tv0):
        dsc.wait()
    for dsc in out_descs(NCHUNK - 1, tv1):
        dsc.wait()


def kernel(x, rho, indices, thres):
    t = jax.nn.softplus(thres[0]) / rho[0]
    t16 = jnp.full((16,), t, dtype=jnp.float32)
    idx = indices.astype(jnp.int32).reshape(NB, 128)
    xf = x.transpose(1, 0).reshape(2, 8, NB, 128).transpose(0, 2, 1, 3).reshape(-1)
    xrm = _transpose_sc(xf)
    outf = _gather_shrink_sc(xrm, idx, t16)
    return (outf.reshape(2, NB, 8, 128).transpose(1, 3, 0, 2).reshape(N, D))


# speedup vs baseline: 1.7440x; 1.5281x over previous
import functools

import jax
import jax.numpy as jnp
from jax import lax
from jax.experimental import pallas as pl
from jax.experimental.pallas import tpu as pltpu
from jax.experimental.pallas import tpu_sc as plsc

N = 2097152
D = 16
NB = N // 128
NC = 2
NS = 16
NW = NC * NS
C = 1024
TB = C // 128
G = C // 128
RW = N // NW
NCHUNK = RW // C
U = 8
PLANE = TB * 8 * 128
HPLANE = NB * 8 * 128

_mesh = plsc.VectorSubcoreMesh(core_axis_name="c", subcore_axis_name="s")


def _flat_consts():
    l = jax.lax.iota(jnp.int32, 16)
    a_c = l >> 3
    f_c = l & 7
    return a_c * PLANE + f_c * 128


@functools.partial(
    pl.kernel,
    mesh=_mesh,
    compiler_params=pltpu.CompilerParams(use_tc_tiling_on_sc=False,
                                         needs_layout_passes=False),
    out_type=jax.ShapeDtypeStruct((N, D), jnp.float32),
    scratch_types=[
        pltpu.VMEM((2 * PLANE,), jnp.float32),
        pltpu.VMEM((2 * PLANE,), jnp.float32),
        pltpu.VMEM((C, D), jnp.float32),
        pltpu.VMEM((C, D), jnp.float32),
        pltpu.SemaphoreType.DMA,
        pltpu.SemaphoreType.DMA,
    ],
)
def _transpose_sc(xf_hbm, xrm_hbm, tb0, tb1, rv0, rv1, sin, sout):
    wid = lax.axis_index("s") * NC + lax.axis_index("c")
    b_base = wid * (NB // NW)
    base_c = _flat_consts()

    def in_descs(jc, tb):
        b0 = b_base + jc * TB
        return (
            pltpu.make_async_copy(xf_hbm.at[pl.ds(b0 * 1024, PLANE)],
                                  tb.at[pl.ds(0, PLANE)], sin),
            pltpu.make_async_copy(xf_hbm.at[pl.ds(HPLANE + b0 * 1024, PLANE)],
                                  tb.at[pl.ds(PLANE, PLANE)], sin),
        )

    def out_desc(jc, rv):
        b0 = b_base + jc * TB
        return pltpu.make_async_copy(rv, xrm_hbm.at[pl.ds(b0 * 128, C)], sout)

    for dsc in in_descs(0, tb0):
        dsc.start()
    for dsc in in_descs(1, tb1):
        dsc.start()

    def body(jj, carry):
        for b, tb, rv in ((0, tb0, rv0), (1, tb1, rv1)):
            j = jj * 2 + b
            for dsc in in_descs(j, tb):
                dsc.wait()

            @pl.when(jj > 0)
            def _():
                out_desc(j - 2, rv).wait()

            def tile_body(bb, carry2):
                iv0 = base_c + jnp.broadcast_to(bb * 1024, (16,))
                rb0 = bb * 128

                def r_body(ii, iv):
                    rbase = rb0 + ii * U
                    vals = [plsc.load_gather(tb, [iv | u]) for u in range(U)]
                    for u in range(U):
                        rv[rbase + u] = vals[u]
                    return iv + U

                lax.fori_loop(0, 128 // U, r_body, iv0)
                return carry2

            lax.fori_loop(0, TB, tile_body, 0)
            out_desc(j, rv).start()
            nj = jnp.where(j + 2 < NCHUNK, j + 2, 0)
            for dsc in in_descs(nj, tb):
                dsc.start()
        return carry

    lax.fori_loop(0, NCHUNK // 2, body, 0)
    for dsc in in_descs(0, tb0):
        dsc.wait()
    for dsc in in_descs(0, tb1):
        dsc.wait()
    out_desc(NCHUNK - 2, rv0).wait()
    out_desc(NCHUNK - 1, rv1).wait()


@functools.partial(
    pl.kernel,
    mesh=_mesh,
    compiler_params=pltpu.CompilerParams(use_tc_tiling_on_sc=False,
                                         needs_layout_passes=False),
    out_type=jax.ShapeDtypeStruct((2 * HPLANE,), jnp.float32),
    scratch_types=[
        pltpu.VMEM((G, 128), jnp.int32),
        pltpu.VMEM((G, 128), jnp.int32),
        pltpu.VMEM((C, D), jnp.float32),
        pltpu.VMEM((C, D), jnp.float32),
        pltpu.VMEM((C, D), jnp.float32),
        pltpu.VMEM((C, D), jnp.float32),
        pltpu.VMEM((2 * PLANE,), jnp.float32),
        pltpu.VMEM((2 * PLANE,), jnp.float32),
        pltpu.VMEM((16,), jnp.float32),
        pltpu.SemaphoreType.DMA,
        pltpu.SemaphoreType.DMA,
        pltpu.SemaphoreType.DMA,
        pltpu.SemaphoreType.DMA,
    ],
)
def _gather_shrink_sc(xrm_hbm, idx_hbm, t_hbm, outf_hbm,
                      iv0_, iv1_, rv0, rv1, xv0, xv1, tv0, tv1, t_v,
                      sidx, sg, sx, sout):
    wid = lax.axis_index("s") * NC + lax.axis_index("c")
    base = wid * RW
    base_c = _flat_consts()
    pltpu.sync_copy(t_hbm, t_v)
    tvec = t_v[...]
    signbit = jnp.broadcast_to(jnp.int32(-2147483648), (16,))
    zero = jnp.zeros((16,), jnp.float32)

    def idx_desc(jc, ivb):
        return pltpu.make_async_copy(
            idx_hbm.at[pl.ds(wid * (RW // 128) + jc * G, G)], ivb, sidx)

    def gather_descs(jc, ivb, rvb):
        return [
            pltpu.make_async_copy(xrm_hbm.at[ivb.at[g]],
                                  rvb.at[pl.ds(g * 128, 128)], sg)
            for g in range(G)
        ]

    def x_desc(jc, xvb):
        return pltpu.make_async_copy(
            xrm_hbm.at[pl.ds(base + jc * C, C)], xvb, sx)

    def out_descs(jc, tvb):
        off8 = (base + jc * C) * 8
        return (
            pltpu.make_async_copy(tvb.at[pl.ds(0, PLANE)],
                                  outf_hbm.at[pl.ds(off8, PLANE)], sout),
            pltpu.make_async_copy(tvb.at[pl.ds(PLANE, PLANE)],
                                  outf_hbm.at[pl.ds(HPLANE + off8, PLANE)],
                                  sout),
        )

    bufs = ((iv0_, rv0, xv0, tv0), (iv1_, rv1, xv1, tv1))

    idx_desc(0, iv0_).start()
    idx_desc(0, iv0_).wait()
    for dsc in gather_descs(0, iv0_, rv0):
        dsc.start()
    x_desc(0, xv0).start()
    idx_desc(1, iv1_).start()

    def body(jj, carry):
        for b in (0, 1):
            ivb, rvb, xvb, tvb = bufs[b]
            nivb, nrvb, nxvb, _ = bufs[1 - b]
            j = jj * 2 + b
            for dsc in gather_descs(j, ivb, rvb):
                dsc.wait()
            x_desc(j, xvb).wait()

            @pl.when(jj > 0)
            def _():
                for dsc in out_descs(j - 2, tvb):
                    dsc.wait()

            def tile_body(bb, carry2):
                iv = base_c + jnp.broadcast_to(bb * 1024, (16,))
                rb0 = bb * 128

                def r_body(ii, ivv):
                    rbase = rb0 + ii * U
                    gvs = [rvb[rbase + u] for u in range(U)]
                    xvs = [xvb[rbase + u] for u in range(U)]
                    for u in range(U):
                        gv = gvs[u]
                        xv = xvs[u]
                        s = jnp.maximum(jnp.abs(gv) - tvec, 0.0)
                        zb = plsc.bitcast(s, jnp.int32) | (
                            plsc.bitcast(xv, jnp.int32) & signbit)
                        z = jnp.where(xv == 0.0, zero,
                                      plsc.bitcast(zb, jnp.float32))
                        plsc.store_scatter(tvb, [ivv | u], z)
                    return ivv + U

                lax.fori_loop(0, 128 // U, r_body, iv)
                return carry2

            lax.fori_loop(0, TB, tile_body, 0)
            for dsc in out_descs(j, tvb):
                dsc.start()
            nj = jnp.where(j + 1 < NCHUNK, j + 1, 0)
            idx_desc(nj, nivb).wait()
            for dsc in gather_descs(nj, nivb, nrvb):
                dsc.start()
            x_desc(nj, nxvb).start()
            nj2 = jnp.where(j + 2 < NCHUNK, j + 2, 0)
            idx_desc(nj2, ivb).start()
        return carry

    lax.fori_loop(0, NCHUNK // 2, body, 0)
    for dsc in gather_descs(0, iv0_, rv0):
        dsc.wait()
    x_desc(0, xv0).wait()
    idx_desc(0, iv1_).wait()
    for dsc in out_descs(NCHUNK - 2, tv0):
        dsc.wait()
    for dsc in out_descs(NCHUNK - 1, tv1):
        dsc.wait()


def kernel(x, rho, indices, thres):
    t = jax.nn.softplus(thres[0]) / rho[0]
    t16 = jnp.full((16,), t, dtype=jnp.float32)
    idx = indices.astype(jnp.int32).reshape(NB, 128)
    xf = x.transpose(1, 0).reshape(2, 8, NB, 128).transpose(0, 2, 1, 3).reshape(-1)
    xrm = _transpose_sc(xf)
    outf = _gather_shrink_sc(xrm, idx, t16)
    return (outf.reshape(2, NB, 8, 128).transpose(1, 3, 0, 2).reshape(N, D))


# speedup vs baseline: 4.2426x; 2.4327x over previous
import functools

import jax
import jax.numpy as jnp
from jax import lax
from jax.experimental import pallas as pl
from jax.experimental.pallas import tpu as pltpu
from jax.experimental.pallas import tpu_sc as plsc

N = 2097152
D = 16
NB = N // 128
NC = 2
NS = 16
NW = NC * NS
C = 1024
TB = C // 128
G = C // 128
RW = N // NW
NCHUNK = RW // C
U = 8
PITCH = 129

_mesh = plsc.VectorSubcoreMesh(core_axis_name="c", subcore_axis_name="s")


def _lane_consts():
    l = jax.lax.iota(jnp.int32, 16)
    a_c = l >> 3
    f_c = l & 7
    return a_c, f_c


@functools.partial(
    pl.kernel,
    mesh=_mesh,
    compiler_params=pltpu.CompilerParams(use_tc_tiling_on_sc=False,
                                         needs_layout_passes=False),
    out_type=jax.ShapeDtypeStruct((N, D), jnp.float32),
    scratch_types=[
        pltpu.VMEM((2, TB * 8, PITCH), jnp.float32),
        pltpu.VMEM((2, TB * 8, PITCH), jnp.float32),
        pltpu.VMEM((C, D), jnp.float32),
        pltpu.VMEM((C, D), jnp.float32),
        pltpu.SemaphoreType.DMA,
        pltpu.SemaphoreType.DMA,
    ],
)
def _transpose_sc(xf_hbm, xrm_hbm, tb0, tb1, rv0, rv1, sin, sout):
    wid = lax.axis_index("s") * NC + lax.axis_index("c")
    b_base = wid * (NB // NW)
    a_c, f_c = _lane_consts()

    def in_descs(jc, tb):
        r0 = (b_base + jc * TB) * 8
        return [
            pltpu.make_async_copy(xf_hbm.at[a, pl.ds(r0, TB * 8)],
                                  tb.at[a, :, pl.ds(0, 128)], sin)
            for a in (0, 1)
        ]

    def out_desc(jc, rv):
        b0 = b_base + jc * TB
        return pltpu.make_async_copy(rv, xrm_hbm.at[pl.ds(b0 * 128, C)], sout)

    for dsc in in_descs(0, tb0):
        dsc.start()
    for dsc in in_descs(1, tb1):
        dsc.start()

    def body(jj, carry):
        for b, tb, rv in ((0, tb0, rv0), (1, tb1, rv1)):
            j = jj * 2 + b
            for dsc in in_descs(j, tb):
                dsc.wait()

            @pl.when(jj > 0)
            def _():
                out_desc(j - 2, rv).wait()

            def tile_body(bb, carry2):
                bfv = f_c + jnp.broadcast_to(bb * 8, (16,))
                rb0 = bb * 128
                iv0 = jnp.zeros((16,), jnp.int32)

                def r_body(ii, iv):
                    rbase = rb0 + ii * U
                    vals = [plsc.load_gather(tb, [a_c, bfv, iv | u])
                            for u in range(U)]
                    for u in range(U):
                        rv[rbase + u] = vals[u]
                    return iv + U

                lax.fori_loop(0, 128 // U, r_body, iv0)
                return carry2

            lax.fori_loop(0, TB, tile_body, 0)
            out_desc(j, rv).start()
            nj = jnp.where(j + 2 < NCHUNK, j + 2, 0)
            for dsc in in_descs(nj, tb):
                dsc.start()
        return carry

    lax.fori_loop(0, NCHUNK // 2, body, 0)
    for dsc in in_descs(0, tb0):
        dsc.wait()
    for dsc in in_descs(0, tb1):
        dsc.wait()
    out_desc(NCHUNK - 2, rv0).wait()
    out_desc(NCHUNK - 1, rv1).wait()


@functools.partial(
    pl.kernel,
    mesh=_mesh,
    compiler_params=pltpu.CompilerParams(use_tc_tiling_on_sc=False,
                                         needs_layout_passes=False),
    out_type=jax.ShapeDtypeStruct((2, NB * 8, 128), jnp.float32),
    scratch_types=[
        pltpu.VMEM((G, 128), jnp.int32),
        pltpu.VMEM((G, 128), jnp.int32),
        pltpu.VMEM((C, D), jnp.float32),
        pltpu.VMEM((C, D), jnp.float32),
        pltpu.VMEM((C, D), jnp.float32),
        pltpu.VMEM((C, D), jnp.float32),
        pltpu.VMEM((2, TB * 8, PITCH), jnp.float32),
        pltpu.VMEM((2, TB * 8, PITCH), jnp.float32),
        pltpu.VMEM((16,), jnp.float32),
        pltpu.SemaphoreType.DMA,
        pltpu.SemaphoreType.DMA,
        pltpu.SemaphoreType.DMA,
        pltpu.SemaphoreType.DMA,
    ],
)
def _gather_shrink_sc(xrm_hbm, idx_hbm, t_hbm, outf_hbm,
                      iv0_, iv1_, rv0, rv1, xv0, xv1, tv0, tv1, t_v,
                      sidx, sg, sx, sout):
    wid = lax.axis_index("s") * NC + lax.axis_index("c")
    base = wid * RW
    a_c, f_c = _lane_consts()
    pltpu.sync_copy(t_hbm, t_v)
    tvec = t_v[...]
    signbit = jnp.broadcast_to(jnp.int32(-2147483648), (16,))
    zero = jnp.zeros((16,), jnp.float32)

    def idx_desc(jc, ivb):
        return pltpu.make_async_copy(
            idx_hbm.at[pl.ds(wid * (RW // 128) + jc * G, G)], ivb, sidx)

    def gather_descs(jc, ivb, rvb):
        return [
            pltpu.make_async_copy(xrm_hbm.at[ivb.at[g]],
                                  rvb.at[pl.ds(g * 128, 128)], sg)
            for g in range(G)
        ]

    def x_desc(jc, xvb):
        return pltpu.make_async_copy(
            xrm_hbm.at[pl.ds(base + jc * C, C)], xvb, sx)

    def out_descs(jc, tvb):
        r0 = (base + jc * C) // 16
        return [
            pltpu.make_async_copy(tvb.at[a, :, pl.ds(0, 128)],
                                  outf_hbm.at[a, pl.ds(r0, TB * 8)], sout)
            for a in (0, 1)
        ]

    bufs = ((iv0_, rv0, xv0, tv0), (iv1_, rv1, xv1, tv1))

    idx_desc(0, iv0_).start()
    idx_desc(0, iv0_).wait()
    for dsc in gather_descs(0, iv0_, rv0):
        dsc.start()
    x_desc(0, xv0).start()
    idx_desc(1, iv1_).start()

    def body(jj, carry):
        for b in (0, 1):
            ivb, rvb, xvb, tvb = bufs[b]
            nivb, nrvb, nxvb, _ = bufs[1 - b]
            j = jj * 2 + b
            for dsc in gather_descs(j, ivb, rvb):
                dsc.wait()
            x_desc(j, xvb).wait()

            @pl.when(jj > 0)
            def _():
                for dsc in out_descs(j - 2, tvb):
                    dsc.wait()

            def tile_body(bb, carry2):
                bfv = f_c + jnp.broadcast_to(bb * 8, (16,))
                rb0 = bb * 128
                iv0v = jnp.zeros((16,), jnp.int32)

                def r_body(ii, ivv):
                    rbase = rb0 + ii * U
                    gvs = [rvb[rbase + u] for u in range(U)]
                    xvs = [xvb[rbase + u] for u in range(U)]
                    for u in range(U):
                        gv = gvs[u]
                        xv = xvs[u]
                        s = jnp.maximum(jnp.abs(gv) - tvec, 0.0)
                        zb = plsc.bitcast(s, jnp.int32) | (
                            plsc.bitcast(xv, jnp.int32) & signbit)
                        z = jnp.where(xv == 0.0, zero,
                                      plsc.bitcast(zb, jnp.float32))
                        plsc.store_scatter(tvb, [a_c, bfv, ivv | u], z)
                    return ivv + U

                lax.fori_loop(0, 128 // U, r_body, iv0v)
                return carry2

            lax.fori_loop(0, TB, tile_body, 0)
            for dsc in out_descs(j, tvb):
                dsc.start()
            nj = jnp.where(j + 1 < NCHUNK, j + 1, 0)
            idx_desc(nj, nivb).wait()
            for dsc in gather_descs(nj, nivb, nrvb):
                dsc.start()
            x_desc(nj, nxvb).start()
            nj2 = jnp.where(j + 2 < NCHUNK, j + 2, 0)
            idx_desc(nj2, ivb).start()
        return carry

    lax.fori_loop(0, NCHUNK // 2, body, 0)
    for dsc in gather_descs(0, iv0_, rv0):
        dsc.wait()
    x_desc(0, xv0).wait()
    idx_desc(0, iv1_).wait()
    for dsc in out_descs(NCHUNK - 2, tv0):
        dsc.wait()
    for dsc in out_descs(NCHUNK - 1, tv1):
        dsc.wait()


def kernel(x, rho, indices, thres):
    t = jax.nn.softplus(thres[0]) / rho[0]
    t16 = jnp.full((16,), t, dtype=jnp.float32)
    idx = indices.astype(jnp.int32).reshape(NB, 128)
    xf = (x.transpose(1, 0).reshape(2, 8, NB, 128).transpose(0, 2, 1, 3)
          .reshape(2, NB * 8, 128))
    xrm = _transpose_sc(xf)
    outf = _gather_shrink_sc(xrm, idx, t16)
    return (outf.reshape(2, NB, 8, 128).transpose(1, 3, 0, 2).reshape(N, D))
